# Initial kernel scaffold; baseline (speedup 1.0000x reference)
import jax
import jax.numpy as jnp
from jax.experimental import pallas as pl


def kernel(full_node_features, adj_normalized, batch_node_idx, batch_ethnicity_idx, W1, b1, W2, b2, Wo1, bo1, Wo2, bo2):
    raise NotImplementedError("write your pallas kernel here")


# speedup vs baseline: 1.4471x; 1.4471x over previous
import functools

import jax
import jax.numpy as jnp
from jax.experimental import pallas as pl
from jax.experimental.pallas import tpu as pltpu

N_NODES_ = 10000
FEAT = 128


BR1 = 400


def _layer1_kernel(a_ref, x_ref, w1_ref, b1_ref, out_ref):
    s = jnp.dot(a_ref[...], x_ref[...], preferred_element_type=jnp.float32)
    y = jax.lax.dot_general(s, w1_ref[...],
                            (((1,), (1,)), ((), ())),
                            preferred_element_type=jnp.float32)
    out_ref[...] = jnp.maximum(y + b1_ref[...], 0.0)


def _layer1(adj, x, w1, b1_2d):
    grid = (N_NODES_ // BR1,)
    return pl.pallas_call(
        _layer1_kernel,
        grid=grid,
        in_specs=[
            pl.BlockSpec((BR1, N_NODES_), lambda i: (i, 0)),
            pl.BlockSpec((N_NODES_, FEAT), lambda i: (0, 0)),
            pl.BlockSpec((FEAT, FEAT), lambda i: (0, 0)),
            pl.BlockSpec((1, FEAT), lambda i: (0, 0)),
        ],
        out_specs=pl.BlockSpec((BR1, FEAT), lambda i: (i, 0)),
        out_shape=jax.ShapeDtypeStruct((N_NODES_, FEAT), jnp.float32),
        compiler_params=pltpu.CompilerParams(
            dimension_semantics=("arbitrary",),
        ),
    )(adj, x, w1, b1_2d)


RB = 256
NB = 4096 // RB


def _layer2_kernel(idx_ref, a_hbm, x1_ref, w2_ref, b2_ref,
                   wo1_ref, bo1_ref, wo2_ref, bo2_ref,
                   out_ref, rows_ref, sem_ref):
    g = pl.program_id(0)

    def issue(slot, step):
        base = step * RB
        def body(r, _):
            row = idx_ref[base + r]
            pltpu.make_async_copy(
                a_hbm.at[pl.ds(row, 1), :],
                rows_ref.at[slot, pl.ds(r, 1), :],
                sem_ref.at[slot],
            ).start()
            return 0
        jax.lax.fori_loop(0, RB, body, 0, unroll=8)

    def wait(slot):
        pltpu.make_async_copy(
            a_hbm.at[pl.ds(0, RB), :],
            rows_ref.at[slot],
            sem_ref.at[slot],
        ).wait()

    @pl.when(g == 0)
    def _prologue():
        issue(0, 0)

    @pl.when(g < NB - 1)
    def _prefetch_next():
        issue((g + 1) % 2, g + 1)

    slot = g % 2
    wait(slot)

    s2 = jnp.dot(rows_ref[slot], x1_ref[...],
                 preferred_element_type=jnp.float32)
    x2 = jax.lax.dot_general(s2, w2_ref[...], (((1,), (1,)), ((), ())),
                             preferred_element_type=jnp.float32) + b2_ref[...]
    h = jnp.maximum(
        jax.lax.dot_general(x2, wo1_ref[...], (((1,), (1,)), ((), ())),
                            preferred_element_type=jnp.float32)
        + bo1_ref[...], 0.0)
    pt = jax.lax.dot_general(wo2_ref[...], h, (((1,), (1,)), ((), ())),
                             preferred_element_type=jnp.float32)
    out_ref[...] = (pt + bo2_ref[0, 0]).reshape(1, 1, RB)


def _layer2_head(idx, adj, x1, w2, b2_2d, wo1, bo1_2d, wo2, bo2_2d):
    grid_spec = pltpu.PrefetchScalarGridSpec(
        num_scalar_prefetch=1,
        grid=(NB,),
        in_specs=[
            pl.BlockSpec(memory_space=pltpu.MemorySpace.HBM),
            pl.BlockSpec((N_NODES_, FEAT), lambda g, sidx: (0, 0)),
            pl.BlockSpec((FEAT, FEAT), lambda g, sidx: (0, 0)),
            pl.BlockSpec((1, FEAT), lambda g, sidx: (0, 0)),
            pl.BlockSpec((FEAT // 2, FEAT), lambda g, sidx: (0, 0)),
            pl.BlockSpec((1, FEAT // 2), lambda g, sidx: (0, 0)),
            pl.BlockSpec((1, FEAT // 2), lambda g, sidx: (0, 0)),
            pl.BlockSpec(memory_space=pltpu.MemorySpace.SMEM),
        ],
        out_specs=pl.BlockSpec((1, 1, RB), lambda g, sidx: (g, 0, 0)),
        scratch_shapes=[
            pltpu.VMEM((2, RB, N_NODES_), jnp.float32),
            pltpu.SemaphoreType.DMA((2,)),
        ],
    )
    out = pl.pallas_call(
        _layer2_kernel,
        grid_spec=grid_spec,
        out_shape=jax.ShapeDtypeStruct((NB, 1, RB), jnp.float32),
        compiler_params=pltpu.CompilerParams(
            dimension_semantics=("arbitrary",),
        ),
    )(idx, adj, x1, w2, b2_2d, wo1, bo1_2d, wo2, bo2_2d)
    return out.reshape(4096)


def kernel(full_node_features, adj_normalized, batch_node_idx,
           batch_ethnicity_idx, W1, b1, W2, b2, Wo1, bo1, Wo2, bo2):
    x1 = _layer1(adj_normalized, full_node_features, W1,
                 b1.reshape(1, FEAT))
    pred = _layer2_head(
        batch_node_idx.astype(jnp.int32), adj_normalized, x1,
        W2, b2.reshape(1, FEAT),
        Wo1, bo1.reshape(1, FEAT // 2),
        Wo2, bo2.reshape(1, 1),
    )
    return pred
